# Initial kernel scaffold; baseline (speedup 1.0000x reference)
import jax
import jax.numpy as jnp
from jax.experimental import pallas as pl


def kernel(embedding, W1_l, b1, W1_r, W2_l, b2, W2_r, edge_index, edge_label_index):
    raise NotImplementedError("write your pallas kernel here")


# speedup vs baseline: 7.2816x; 7.2816x over previous
import functools

import jax
import jax.numpy as jnp
from jax import lax
from jax.experimental import pallas as pl
from jax.experimental.pallas import tpu as pltpu
from jax.experimental.pallas import tpu_sc as plsc

NC = 2
NS = 16
NW = NC * NS
LN = 16
CH = 40
RD = 4


def _zero_buf(buf, nrows, d):
    def zb(i, _):
        buf[i // (d // LN), pl.ds((i % (d // LN)) * LN, LN)] = jnp.zeros(
            (LN,), jnp.float32)
        return 0
    lax.fori_loop(0, nrows * (d // LN), zb, 0)


def _agg_body(n_nodes, d, n_edges, *refs):
    (table, src, dst, sp_out, acc,
     sidx, didx, rows, sem_si, sem_di, sem_g, sem_s) = refs

    cid = lax.axis_index("c")
    sid = lax.axis_index("s")
    wid = sid * NC + cid

    epw = n_edges // NW
    nch = epw // CH
    nz = n_nodes // CH
    nzt = (nz + NS - 1) // NS

    def idx_start(c, slot):
        base = wid * epw + c * CH
        pltpu.async_copy(src.at[pl.ds(base, CH)], sidx.at[slot],
                         sem_si.at[slot])
        pltpu.async_copy(dst.at[pl.ds(base, CH)], didx.at[slot],
                         sem_di.at[slot])

    def idx_wait(slot):
        pltpu.make_async_copy(src.at[pl.ds(0, CH)], sidx.at[slot],
                              sem_si.at[slot]).wait()
        pltpu.make_async_copy(dst.at[pl.ds(0, CH)], didx.at[slot],
                              sem_di.at[slot]).wait()

    def gather_start(slot):
        pltpu.async_copy(table.at[sidx.at[slot]], rows.at[slot],
                         sem_g.at[slot])

    def gather_wait(slot):
        pltpu.make_async_copy(table.at[sidx.at[slot]], rows.at[slot],
                              sem_g.at[slot]).wait()

    def scat_start(slot):
        pltpu.async_copy(rows.at[slot], acc.at[didx.at[slot]],
                         sem_s.at[slot], add=True)

    def scat_wait(slot):
        pltpu.make_async_copy(rows.at[slot], acc.at[didx.at[slot]],
                              sem_s.at[slot]).wait()

    _zero_buf(rows.at[0], CH, d)
    for j in range(nzt):
        c = sid + NS * j

        @pl.when(c < nz)
        def _():
            pltpu.sync_copy(rows.at[0], acc.at[pl.ds(c * CH, CH)])

    plsc.subcore_barrier()

    idx_start(0, 0)
    idx_start(1, 1)

    def body(c, _):
        slot = lax.rem(c, RD)
        prev = lax.rem(c + RD - 1, RD)
        pslot = lax.rem(c + RD - 2, RD)

        @pl.when(c >= 2)
        def _():
            scat_wait(pslot)
        idx_wait(slot)
        gather_start(slot)

        @pl.when(c >= 1)
        def _():
            gather_wait(prev)
            scat_start(prev)

        @pl.when(c + 2 < nch)
        def _():
            idx_start(c + 2, pslot)
        return 0
    lax.fori_loop(0, nch, body, 0)

    last = lax.rem(nch - 1, RD)
    gather_wait(last)
    scat_start(last)
    scat_wait(lax.rem(nch - 2, RD))
    scat_wait(last)

    plsc.subcore_barrier()

    for j in range(nzt):
        c = sid + NS * j

        @pl.when(c < nz)
        def _():
            pltpu.sync_copy(acc.at[pl.ds(c * CH, CH)], rows.at[0])
            pltpu.sync_copy(rows.at[0], sp_out.at[cid, pl.ds(c * CH, CH)])


def _make_agg(n_nodes, d, n_edges):
    scratch = (
        pltpu.VMEM_SHARED((n_nodes, d), jnp.float32),
        pltpu.VMEM((RD, CH), jnp.int32),
        pltpu.VMEM((RD, CH), jnp.int32),
        pltpu.VMEM((RD, CH, d), jnp.float32),
        pltpu.SemaphoreType.DMA((RD,)),
        pltpu.SemaphoreType.DMA((RD,)),
        pltpu.SemaphoreType.DMA((RD,)),
        pltpu.SemaphoreType.DMA((RD,)),
    )
    return pl.kernel(
        functools.partial(_agg_body, n_nodes, d, n_edges),
        out_type=jax.ShapeDtypeStruct((NC, n_nodes, d), jnp.float32),
        mesh=plsc.VectorSubcoreMesh(core_axis_name="c", subcore_axis_name="s"),
        scratch_types=scratch,
        name="sage_agg",
    )


def _deg_body(n_nodes, d, n_edges, dst, deg_out,
              dacc, didx, ones_v, sem_di, sem_s):
    cid = lax.axis_index("c")
    sid = lax.axis_index("s")
    wid = sid * NC + cid

    epw = n_edges // NW
    nch = epw // CH
    nz = n_nodes // CH
    nzt = (nz + NS - 1) // NS

    def idx_start(c, slot):
        base = wid * epw + c * CH
        pltpu.async_copy(dst.at[pl.ds(base, CH)], didx.at[slot],
                         sem_di.at[slot])

    def idx_wait(slot):
        pltpu.make_async_copy(dst.at[pl.ds(0, CH)], didx.at[slot],
                              sem_di.at[slot]).wait()

    def scat_start(slot):
        pltpu.async_copy(ones_v, dacc.at[didx.at[slot]],
                         sem_s.at[slot], add=True)

    def scat_wait(slot):
        pltpu.make_async_copy(ones_v, dacc.at[didx.at[slot]],
                              sem_s.at[slot]).wait()

    _zero_buf(ones_v, CH, d)
    for j in range(nzt):
        c = sid + NS * j

        @pl.when(c < nz)
        def _():
            pltpu.sync_copy(ones_v, dacc.at[pl.ds(c * CH, CH)])

    def zo(i, _):
        for q in range(d // LN):
            ones_v[i, pl.ds(q * LN, LN)] = jnp.full((LN,), 1.0, jnp.float32)
        return 0
    lax.fori_loop(0, CH, zo, 0)

    plsc.subcore_barrier()

    idx_start(0, 0)
    idx_start(1, 1)

    def body(c, _):
        slot = lax.rem(c, RD)
        pslot = lax.rem(c + RD - 2, RD)

        @pl.when(c >= 2)
        def _():
            scat_wait(pslot)
        idx_wait(slot)
        scat_start(slot)

        @pl.when(c + 2 < nch)
        def _():
            idx_start(c + 2, pslot)
        return 0
    lax.fori_loop(0, nch, body, 0)

    scat_wait(lax.rem(nch - 2, RD))
    scat_wait(lax.rem(nch - 1, RD))

    plsc.subcore_barrier()

    for j in range(nzt):
        c = sid + NS * j

        @pl.when(c < nz)
        def _():
            pltpu.sync_copy(dacc.at[pl.ds(c * CH, CH)], ones_v)
            pltpu.sync_copy(ones_v, deg_out.at[cid, pl.ds(c * CH, CH)])


def _make_deg(n_nodes, d, n_edges):
    return pl.kernel(
        functools.partial(_deg_body, n_nodes, d, n_edges),
        out_type=jax.ShapeDtypeStruct((NC, n_nodes, d), jnp.float32),
        mesh=plsc.VectorSubcoreMesh(core_axis_name="c", subcore_axis_name="s"),
        scratch_types=(
            pltpu.VMEM_SHARED((n_nodes, d), jnp.float32),
            pltpu.VMEM((RD, CH), jnp.int32),
            pltpu.VMEM((CH, d), jnp.float32),
            pltpu.SemaphoreType.DMA((RD,)),
            pltpu.SemaphoreType.DMA((RD,)),
        ),
        name="deg_hist",
    )


def _combine_block(relu, sp_ref, degp_ref, x_ref, wl_ref, b_ref, wr_ref,
                   o_ref):
    s = sp_ref[0] + sp_ref[1]
    deg = degp_ref[0, :, 0] + degp_ref[1, :, 0]
    agg = s / jnp.maximum(deg, 1.0)[:, None]
    dn = (((1,), (1,)), ((), ()))
    y = lax.dot_general(agg, wl_ref[...], dn,
                        precision=lax.Precision.HIGHEST) + b_ref[...]
    y = y + lax.dot_general(x_ref[...], wr_ref[...], dn,
                            precision=lax.Precision.HIGHEST)
    o_ref[...] = jnp.maximum(y, 0.0) if relu else y


def _combine(sp, degp, x, wl, b, wr, relu):
    n, d = x.shape
    br = 1000
    grid = (n // br,)
    return pl.pallas_call(
        functools.partial(_combine_block, relu),
        grid=grid,
        in_specs=[
            pl.BlockSpec((NC, br, d), lambda i: (0, i, 0)),
            pl.BlockSpec((NC, br, d), lambda i: (0, i, 0)),
            pl.BlockSpec((br, d), lambda i: (i, 0)),
            pl.BlockSpec((d, d), lambda i: (0, 0)),
            pl.BlockSpec((1, d), lambda i: (0, 0)),
            pl.BlockSpec((d, d), lambda i: (0, 0)),
        ],
        out_specs=pl.BlockSpec((br, d), lambda i: (i, 0)),
        out_shape=jax.ShapeDtypeStruct((n, d), jnp.float32),
    )(sp, degp, x, wl, b.reshape(1, d), wr)


def _decode_body(d, lp, z, e0, e1, out, i0, i1, s_v, d_v, ob,
                 sem_g, sem_o):
    cid = lax.axis_index("c")
    sid = lax.axis_index("s")
    wid = sid * NC + cid
    PC = 128
    ppw = lp // NW
    T = ppw // PC

    def idx_copy(t, slot):
        base = wid * ppw + t * PC
        pltpu.sync_copy(e0.at[pl.ds(base, PC)], i0.at[slot])
        pltpu.sync_copy(e1.at[pl.ds(base, PC)], i1.at[slot])

    def gather_start(slot):
        pltpu.async_copy(z.at[i0.at[slot]], s_v.at[slot], sem_g.at[slot])
        pltpu.async_copy(z.at[i1.at[slot]], d_v.at[slot], sem_g.at[slot])

    def gather_wait(slot):
        pltpu.make_async_copy(z.at[i0.at[slot]], s_v.at[slot],
                              sem_g.at[slot]).wait()
        pltpu.make_async_copy(z.at[i1.at[slot]], d_v.at[slot],
                              sem_g.at[slot]).wait()

    idx_copy(0, 0)
    gather_start(0)

    lane = lax.iota(jnp.int32, LN)

    for t in range(T):
        slot = t % 2
        nslot = 1 - slot
        if t + 1 < T:
            idx_copy(t + 1, nslot)
            gather_start(nslot)
        gather_wait(slot)
        if t >= 2:
            pltpu.make_async_copy(
                ob.at[slot], out.at[pl.ds(0, PC)], sem_o.at[slot]).wait()

        def group(g, _):
            res = jnp.zeros((LN,), jnp.float32)
            for lidx in range(LN):
                p = g * LN + lidx
                acc = s_v[slot, p, pl.ds(0, LN)] * d_v[slot, p, pl.ds(0, LN)]
                for k in range(1, d // LN):
                    acc = acc + (s_v[slot, p, pl.ds(k * LN, LN)] *
                                 d_v[slot, p, pl.ds(k * LN, LN)])
                for sh in (8, 4, 2, 1):
                    acc = acc + acc.at[lane ^ sh].get(
                        mode="promise_in_bounds")
                res = jnp.where(lane == lidx, acc, res)
            ob[slot, pl.ds(g * LN, LN)] = res
            return 0
        lax.fori_loop(0, PC // LN, group, 0)

        base = wid * ppw + t * PC
        pltpu.async_copy(ob.at[slot], out.at[pl.ds(base, PC)],
                         sem_o.at[slot])

    for t in (T - 2, T - 1):
        pltpu.make_async_copy(
            ob.at[t % 2], out.at[pl.ds(0, PC)], sem_o.at[t % 2]).wait()


def _make_decode(n_nodes, d, lp):
    return pl.kernel(
        functools.partial(_decode_body, d, lp),
        out_type=jax.ShapeDtypeStruct((lp,), jnp.float32),
        mesh=plsc.VectorSubcoreMesh(core_axis_name="c", subcore_axis_name="s"),
        scratch_types=(
            pltpu.VMEM((2, 128), jnp.int32),
            pltpu.VMEM((2, 128), jnp.int32),
            pltpu.VMEM((2, 128, d), jnp.float32),
            pltpu.VMEM((2, 128, d), jnp.float32),
            pltpu.VMEM((2, 128), jnp.float32),
            pltpu.SemaphoreType.DMA((2,)),
            pltpu.SemaphoreType.DMA((2,)),
        ),
        name="decode_dot",
    )


def kernel(embedding, W1_l, b1, W1_r, W2_l, b2, W2_r, edge_index,
           edge_label_index):
    n, d = embedding.shape
    e = edge_index.shape[1]
    l = edge_label_index.shape[1]

    src = edge_index[0]
    dst = edge_index[1]

    degp = _make_deg(n, d, e)(dst)
    s1p = _make_agg(n, d, e)(embedding, src, dst)
    h = _combine(s1p, degp, embedding, W1_l, b1, W1_r, relu=True)
    s2p = _make_agg(n, d, e)(h, src, dst)
    z = _combine(s2p, degp, h, W2_l, b2, W2_r, relu=False)

    lp = ((l + 128 * NW - 1) // (128 * NW)) * (128 * NW)
    e0 = jnp.pad(edge_label_index[0], (0, lp - l))
    e1 = jnp.pad(edge_label_index[1], (0, lp - l))
    out = _make_decode(n, d, lp)(z, e0, e1)
    return out[:l]
